# baseline (speedup 1.0000x reference)
import functools

import jax
import jax.numpy as jnp
from jax import lax
from jax.experimental import pallas as pl
from jax.experimental.pallas import tpu as pltpu
from jax.experimental.pallas import tpu_sc as plsc

NUM_USERS = 100000
NUM_ITEMS = 1000000
EMBED_DIM = 64
HIDDEN_DIM = 128
BATCH = 16384


def _sc_gather(user_ids, item_ids, user_table, item_table):
    info = plsc.get_sparse_core_info()
    nc, ns = info.num_cores, info.num_subcores
    nw = nc * ns
    b_per_w = BATCH // nw

    mesh = plsc.VectorSubcoreMesh(core_axis_name="c", subcore_axis_name="s")

    @functools.partial(
        pl.kernel,
        out_type=(
            jax.ShapeDtypeStruct((BATCH, EMBED_DIM), jnp.float32),
            jax.ShapeDtypeStruct((BATCH, EMBED_DIM), jnp.float32),
        ),
        mesh=mesh,
        compiler_params=pltpu.CompilerParams(use_tc_tiling_on_sc=False),
        scratch_types=[
            pltpu.VMEM((b_per_w,), jnp.int32),
            pltpu.VMEM((b_per_w,), jnp.int32),
            pltpu.VMEM((b_per_w, EMBED_DIM), jnp.float32),
            pltpu.VMEM((b_per_w, EMBED_DIM), jnp.float32),
            pltpu.SemaphoreType.DMA,
            pltpu.SemaphoreType.DMA,
            pltpu.SemaphoreType.DMA,
        ],
    )
    def gather_kernel(uids_hbm, iids_hbm, utab_hbm, itab_hbm, uout_hbm,
                      iout_hbm, uidx_v, iidx_v, urows_v, irows_v,
                      sem_u, sem_i, sem_w):
        wid = lax.axis_index("s") * nc + lax.axis_index("c")
        base = wid * b_per_w
        pltpu.sync_copy(uids_hbm.at[pl.ds(base, b_per_w)], uidx_v)
        pltpu.sync_copy(iids_hbm.at[pl.ds(base, b_per_w)], iidx_v)
        u_dma = pltpu.async_copy(utab_hbm.at[uidx_v], urows_v, sem_u)
        i_dma = pltpu.async_copy(itab_hbm.at[iidx_v], irows_v, sem_i)
        u_dma.wait()
        uw_dma = pltpu.async_copy(urows_v, uout_hbm.at[pl.ds(base, b_per_w)],
                                  sem_w)
        i_dma.wait()
        iw_dma = pltpu.async_copy(irows_v, iout_hbm.at[pl.ds(base, b_per_w)],
                                  sem_w)
        uw_dma.wait()
        iw_dma.wait()

    return gather_kernel(user_ids, item_ids, user_table, item_table)


def _mlp_block(u_ref, i_ref, w1u_ref, w1i_ref, b1_ref, w2_ref, b2_ref,
               out_ref):
    h = (
        jnp.dot(u_ref[...], w1u_ref[...], preferred_element_type=jnp.float32)
        + jnp.dot(i_ref[...], w1i_ref[...], preferred_element_type=jnp.float32)
        + b1_ref[...]
    )
    h = jnp.maximum(h, 0.0)
    out_ref[...] = (
        jnp.sum(h * w2_ref[...], axis=1, keepdims=True) + b2_ref[...]
    )


def _tc_mlp(user_emb, item_emb, w1u_t, w1i_t, b1_row, w2_row, b2_s):
    blk = 2048
    grid = (BATCH // blk,)
    return pl.pallas_call(
        _mlp_block,
        grid=grid,
        in_specs=[
            pl.BlockSpec((blk, EMBED_DIM), lambda i: (i, 0)),
            pl.BlockSpec((blk, EMBED_DIM), lambda i: (i, 0)),
            pl.BlockSpec((EMBED_DIM, HIDDEN_DIM), lambda i: (0, 0)),
            pl.BlockSpec((EMBED_DIM, HIDDEN_DIM), lambda i: (0, 0)),
            pl.BlockSpec((1, HIDDEN_DIM), lambda i: (0, 0)),
            pl.BlockSpec((1, HIDDEN_DIM), lambda i: (0, 0)),
            pl.BlockSpec((1, 1), lambda i: (0, 0)),
        ],
        out_specs=pl.BlockSpec((blk, 1), lambda i: (i, 0)),
        out_shape=jax.ShapeDtypeStruct((BATCH, 1), jnp.float32),
    )(user_emb, item_emb, w1u_t, w1i_t, b1_row, w2_row, b2_s)


def kernel(user_ids, item_ids, user_table, item_table, W1, b1, W2, b2):
    user_emb, item_emb = _sc_gather(
        user_ids.astype(jnp.int32), item_ids.astype(jnp.int32),
        user_table, item_table)
    w1u_t = W1[:, :EMBED_DIM].T
    w1i_t = W1[:, EMBED_DIM:].T
    b1_row = b1.reshape(1, HIDDEN_DIM)
    w2_row = W2.reshape(1, HIDDEN_DIM)
    b2_s = b2.reshape(1, 1)
    return _tc_mlp(user_emb, item_emb, w1u_t, w1i_t, b1_row, w2_row, b2_s)
